# baseline (speedup 1.0000x reference)
import functools
import jax
import jax.numpy as jnp
from jax import lax
from jax.experimental import pallas as pl
from jax.experimental.pallas import tpu as pltpu
from jax.experimental.pallas import tpu_sc as plsc


def _tc_body(emb_ref, alias_ref, out_ref):
    out_ref[...] = jnp.broadcast_to(emb_ref[...][:, None, :], out_ref.shape)


def kernel(x, pos_embedding):
    S, N = x.shape
    D = pos_embedding.shape[1]
    H = 2048
    info = plsc.get_sparse_core_info()
    NC = info.num_cores
    NW = NC * info.num_subcores
    R = (S - H) // NW

    mesh = plsc.VectorSubcoreMesh(core_axis_name="c", subcore_axis_name="s")

    @functools.partial(
        pl.kernel,
        out_type=jax.ShapeDtypeStruct((S, N, D), pos_embedding.dtype),
        mesh=mesh,
        scratch_types=[
            pltpu.VMEM((R, D), pos_embedding.dtype),
            pltpu.SemaphoreType.DMA,
        ],
    )
    def sc_half(table_hbm, out_hbm, buf, sem):
        wid = lax.axis_index("s") * NC + lax.axis_index("c")
        base = H + wid * R
        pltpu.sync_copy(table_hbm.at[pl.ds(base, R)], buf)
        copies = [
            pltpu.async_copy(buf, out_hbm.at[pl.ds(base, R), n], sem)
            for n in range(N)
        ]
        for cp in copies:
            cp.wait()

    partial = sc_half(pos_embedding)

    BS = 512
    return pl.pallas_call(
        _tc_body,
        grid=(H // BS,),
        in_specs=[
            pl.BlockSpec((BS, D), lambda i: (i, 0)),
            pl.BlockSpec(memory_space=pl.ANY),
        ],
        out_specs=pl.BlockSpec((BS, N, D), lambda i: (i, 0, 0)),
        out_shape=jax.ShapeDtypeStruct((S, N, D), pos_embedding.dtype),
        input_output_aliases={1: 0},
    )(pos_embedding, partial)


# speedup vs baseline: 1.0746x; 1.0746x over previous
import functools
import jax
from jax import lax
from jax.experimental import pallas as pl
from jax.experimental.pallas import tpu as pltpu
from jax.experimental.pallas import tpu_sc as plsc


def kernel(x, pos_embedding):
    S, N = x.shape
    D = pos_embedding.shape[1]
    info = plsc.get_sparse_core_info()
    NC = info.num_cores
    NW = NC * info.num_subcores
    R = S // NW

    mesh = plsc.VectorSubcoreMesh(core_axis_name="c", subcore_axis_name="s")

    @functools.partial(
        pl.kernel,
        out_type=jax.ShapeDtypeStruct((S, N, D), pos_embedding.dtype),
        mesh=mesh,
        scratch_types=[
            pltpu.VMEM((R, D), pos_embedding.dtype),
            pltpu.SemaphoreType.DMA,
        ],
    )
    def broadcast_rows(table_hbm, out_hbm, buf, sem):
        wid = lax.axis_index("s") * NC + lax.axis_index("c")
        base = wid * R
        pltpu.sync_copy(table_hbm.at[pl.ds(base, R)], buf)
        copies = [
            pltpu.async_copy(buf, out_hbm.at[pl.ds(base, R), n], sem)
            for n in range(N)
        ]
        for cp in copies:
            cp.wait()

    return broadcast_rows(pos_embedding)
